# Initial kernel scaffold; baseline (speedup 1.0000x reference)
import jax
import jax.numpy as jnp
from jax.experimental import pallas as pl


def kernel(qc_flags, table):
    raise NotImplementedError("write your pallas kernel here")


# speedup vs baseline: 1.4552x; 1.4552x over previous
import functools

import jax
import jax.numpy as jnp
from jax import lax
from jax.experimental import pallas as pl
from jax.experimental.pallas import tpu as pltpu
from jax.experimental.pallas import tpu_sc as plsc

NUM_BITS = 10
EMBED_DIM = 32
VOCAB = 2 ** NUM_BITS
ROWS = 16384
COLS = 100
N = ROWS * COLS

_info = plsc.get_sparse_core_info()
NC = _info.num_cores
NS = _info.num_subcores
L = _info.num_lanes
NW = NC * NS
B_PER_W = N // NW
CHUNK = 1024
N_CHUNKS = B_PER_W // CHUNK


@functools.partial(
    pl.kernel,
    out_type=(
        jax.ShapeDtypeStruct((N,), jnp.float32),
        jax.ShapeDtypeStruct((N, EMBED_DIM), jnp.float32),
        jax.ShapeDtypeStruct((N, NUM_BITS), jnp.float32),
    ),
    mesh=plsc.VectorSubcoreMesh(core_axis_name="c", subcore_axis_name="s"),
    compiler_params=pltpu.CompilerParams(use_tc_tiling_on_sc=False),
    scratch_types=[
        pltpu.VMEM((CHUNK,), jnp.int32),
        pltpu.VMEM((CHUNK, EMBED_DIM), jnp.float32),
        pltpu.VMEM((CHUNK, NUM_BITS), jnp.float32),
        pltpu.VMEM((CHUNK,), jnp.float32),
        pltpu.SemaphoreType.DMA,
    ],
)
def _sc_featurize(flags_hbm, table_hbm, blut_hbm, valid_hbm, emb_hbm,
                  bits_hbm, idx_v, rows_v, bits_v, valid_v, sem):
    wid = lax.axis_index("s") * NC + lax.axis_index("c")
    base0 = wid * B_PER_W

    def chunk_body(c, _):
        base = base0 + c * CHUNK
        pltpu.sync_copy(flags_hbm.at[pl.ds(base, CHUNK)], idx_v)
        g_emb = pltpu.async_copy(table_hbm.at[idx_v], rows_v, sem)
        g_bit = pltpu.async_copy(blut_hbm.at[idx_v], bits_v, sem)

        def vec_body(i, _):
            off = i * L
            flags = idx_v[pl.ds(off, L)]
            valid_v[pl.ds(off, L)] = jnp.where(
                flags == 0, jnp.float32(1.0), jnp.float32(0.0))
            return _

        lax.fori_loop(0, CHUNK // L, vec_body, None)

        g_emb.wait()
        g_bit.wait()
        pltpu.sync_copy(rows_v, emb_hbm.at[pl.ds(base, CHUNK)])
        pltpu.sync_copy(bits_v, bits_hbm.at[pl.ds(base, CHUNK)])
        pltpu.sync_copy(valid_v, valid_hbm.at[pl.ds(base, CHUNK)])
        return _

    lax.fori_loop(0, N_CHUNKS, chunk_body, None)


def _bits_lut():
    v = jnp.arange(VOCAB, dtype=jnp.int32)
    shifts = jnp.arange(NUM_BITS, dtype=jnp.int32)
    return ((v[:, None] >> shifts) & 1).astype(jnp.float32)


def kernel(qc_flags, table):
    flags = qc_flags.reshape(-1).astype(jnp.int32)
    valid, emb, bits = _sc_featurize(flags, table, _bits_lut())
    return (
        valid.reshape(ROWS, COLS),
        emb.reshape(ROWS, COLS, EMBED_DIM),
        bits.reshape(ROWS, COLS, NUM_BITS),
    )


# speedup vs baseline: 18.9118x; 12.9958x over previous
import functools

import jax
import jax.numpy as jnp
from jax import lax
from jax.experimental import pallas as pl
from jax.experimental.pallas import tpu as pltpu
from jax.experimental.pallas import tpu_sc as plsc

NUM_BITS = 10
EMBED_DIM = 32
VOCAB = 2 ** NUM_BITS
ROWS = 16384
COLS = 100
N = ROWS * COLS

_info = plsc.get_sparse_core_info()
NC = _info.num_cores
NS = _info.num_subcores
L = _info.num_lanes
NW = NC * NS
W = ROWS // NW
NV = W // L
CB = COLS // 8
TAIL = COLS - CB * 8


@functools.partial(
    pl.kernel,
    out_type=(
        jax.ShapeDtypeStruct((COLS, ROWS), jnp.float32),
        jax.ShapeDtypeStruct((COLS, EMBED_DIM, ROWS), jnp.float32),
        jax.ShapeDtypeStruct((NUM_BITS, COLS, ROWS), jnp.float32),
    ),
    mesh=plsc.VectorSubcoreMesh(core_axis_name="c", subcore_axis_name="s"),
    compiler_params=pltpu.CompilerParams(
        use_tc_tiling_on_sc=True, needs_layout_passes=False),
    scratch_types=[
        pltpu.VMEM((VOCAB * EMBED_DIM,), jnp.float32),
        pltpu.VMEM((8, W), jnp.int32),
        pltpu.VMEM((EMBED_DIM, W), jnp.float32),
        pltpu.VMEM((NUM_BITS, 8, W), jnp.float32),
        pltpu.VMEM((8, W), jnp.float32),
    ],
)
def _sc_featurize(flags_hbm, ttab_hbm, valid_hbm, emb_hbm, bits_hbm,
                  ttab_v, flg_v, emb_v, bits_v, valid_v):
    wid = lax.axis_index("s") * NC + lax.axis_index("c")
    r0 = wid * W
    pltpu.sync_copy(ttab_hbm, ttab_v)

    one_f = jnp.full((L,), 1.0, jnp.float32)
    zero_f = jnp.zeros((L,), jnp.float32)

    def do_col(cc):
        def vec_body(v, _):
            off = v * L
            flags = flg_v[cc, pl.ds(off, L)]
            valid_v[cc, pl.ds(off, L)] = jnp.where(flags == 0, one_f, zero_f)
            for b in range(NUM_BITS):
                bits_v[b, cc, pl.ds(off, L)] = (
                    (flags >> b) & 1).astype(jnp.float32)
            for d in range(EMBED_DIM):
                emb_v[d, pl.ds(off, L)] = plsc.load_gather(
                    ttab_v, [flags + (d * VOCAB)])
            return _

        lax.fori_loop(0, NV, vec_body, None)

    def blk_body(cb, _):
        c0 = cb * 8
        pltpu.sync_copy(flags_hbm.at[pl.ds(c0, 8), pl.ds(r0, W)], flg_v)
        for cc in range(8):
            do_col(cc)
            pltpu.sync_copy(emb_v, emb_hbm.at[c0 + cc, :, pl.ds(r0, W)])
        for b in range(NUM_BITS):
            pltpu.sync_copy(bits_v.at[b], bits_hbm.at[b, pl.ds(c0, 8), pl.ds(r0, W)])
        pltpu.sync_copy(valid_v, valid_hbm.at[pl.ds(c0, 8), pl.ds(r0, W)])
        return _

    lax.fori_loop(0, CB, blk_body, None)

    c0 = CB * 8
    pltpu.sync_copy(flags_hbm.at[pl.ds(c0, TAIL), pl.ds(r0, W)],
                    flg_v.at[pl.ds(0, TAIL)])
    for cc in range(TAIL):
        do_col(cc)
        pltpu.sync_copy(emb_v, emb_hbm.at[c0 + cc, :, pl.ds(r0, W)])
    for b in range(NUM_BITS):
        pltpu.sync_copy(bits_v.at[b, pl.ds(0, TAIL)],
                        bits_hbm.at[b, pl.ds(c0, TAIL), pl.ds(r0, W)])
    pltpu.sync_copy(valid_v.at[pl.ds(0, TAIL)],
                    valid_hbm.at[pl.ds(c0, TAIL), pl.ds(r0, W)])


def kernel(qc_flags, table):
    flags_t = qc_flags.astype(jnp.int32).T
    ttab = table.T.reshape(-1)
    valid_t, emb_t, bits_t = _sc_featurize(flags_t, ttab)
    return (
        valid_t.T,
        emb_t.transpose(2, 0, 1),
        bits_t.transpose(2, 1, 0),
    )
